# Initial kernel scaffold; baseline (speedup 1.0000x reference)
import jax
import jax.numpy as jnp
from jax.experimental import pallas as pl


def kernel(x, edge_index, edge_attr, batch, params):
    raise NotImplementedError("write your pallas kernel here")


# speedup vs baseline: 2.1302x; 2.1302x over previous
import functools
import jax
import jax.numpy as jnp
from jax import lax
from jax.experimental import pallas as pl
from jax.experimental.pallas import tpu as pltpu
from jax.experimental.pallas import tpu_sc as plsc

N = 10000
E = 320000
HID = 64
EHID = 32
NLAYERS = 4

NC = 2
NS = 16
NW = NC * NS
LANES = 16
EPW = E // NW
CH = 80
NCH = EPW // CH
NP = 10240
RPS = NP // NS
RZ = 128
NRC = RPS // RZ

_f32 = jnp.float32
_sds = jax.ShapeDtypeStruct

_sc_mesh = plsc.VectorSubcoreMesh(core_axis_name="c", subcore_axis_name="s")


def _dot(a, b):
    return jnp.dot(a, b, preferred_element_type=_f32)


def _bn0(t):
    mu = jnp.mean(t, axis=0, keepdims=True)
    d = t - mu
    var = jnp.mean(d * d, axis=0, keepdims=True)
    return d, var


def _node_enc_body(x_ref, w0, b0, g0, be0, w1, b1, wab, h_ref, pq_ref):
    t = _dot(x_ref[...], w0[...]) + b0[...]
    d, var = _bn0(t)
    t = jax.nn.relu(d / jnp.sqrt(var + 1e-5) * g0[...] + be0[...])
    h = _dot(t, w1[...]) + b1[...]
    h_ref[...] = h
    pq_ref[...] = _dot(h, wab[...])


def _estats_body(a_ref, w0, b0, out_ref, acc_ref):
    ph = pl.program_id(0)
    i = pl.program_id(1)
    t = _dot(a_ref[...], w0[...]) + b0[...]

    @pl.when((ph == 0) & (i == 0))
    def _():
        acc_ref[...] = jnp.zeros_like(acc_ref)

    @pl.when(ph == 0)
    def _():
        acc_ref[0:1, :] += jnp.sum(t, axis=0, keepdims=True)

    @pl.when(ph == 1)
    def _():
        d = t - acc_ref[0:1, :] / E
        acc_ref[1:2, :] += jnp.sum(d * d, axis=0, keepdims=True)

    @pl.when((ph == 1) & (i == pl.num_programs(1) - 1))
    def _():
        out_ref[...] = acc_ref[...]


def _redge_body(a_ref, w0, b0, ss, ew1, eb1, w1cs, b1s, r0, r1, r2, r3):
    t = _dot(a_ref[...], w0[...]) + b0[...]
    z = jax.nn.relu(t * ss[0:1, :] + ss[1:2, :])
    e = _dot(z, ew1[...]) + eb1[...]
    outs = (r0, r1, r2, r3)
    for l in range(NLAYERS):
        outs[l][...] = _dot(e, w1cs[32 * l:32 * (l + 1), :]) + b1s[l:l + 1, :]


def _upd_core(h, s0, s1, d0, d1, w2m, b2m, wuh, wua, bu1, gu, beu, wu2, bu2,
              lng, lnb):
    s = s0[...] + s1[...]
    agg = jnp.dot(s, w2m[...], preferred_element_type=_f32,
                  precision=lax.Precision.HIGHEST) \
        + (d0[...] + d1[...]) * b2m[...]
    t = _dot(h, wuh[...]) + _dot(agg, wua[...]) + bu1[...]
    d, var = _bn0(t)
    t = jax.nn.relu(d / jnp.sqrt(var + 1e-5) * gu[...] + beu[...])
    hn = h + _dot(t, wu2[...]) + bu2[...]
    mu = jnp.mean(hn, axis=-1, keepdims=True)
    dn = hn - mu
    var = jnp.mean(dn * dn, axis=-1, keepdims=True)
    return dn / jnp.sqrt(var + 1e-5) * lng[...] + lnb[...]


def _upd_body(h_ref, s0, s1, d0, d1, w2m, b2m, wuh, wua, bu1, gu, beu, wu2,
              bu2, lng, lnb, wab, h_out, pq_out):
    hg = _upd_core(h_ref[...], s0, s1, d0, d1, w2m, b2m, wuh, wua, bu1, gu,
                   beu, wu2, bu2, lng, lnb)
    h_out[...] = hg
    pq_out[...] = _dot(hg, wab[...])


def _upd_head_body(h_ref, s0, s1, d0, d1, w2m, b2m, wuh, wua, bu1, gu, beu,
                   wu2, bu2, lng, lnb, hw0, hb0, hg0, hbe0, hw1, hb1, o_ref):
    hg = _upd_core(h_ref[...], s0, s1, d0, d1, w2m, b2m, wuh, wua, bu1, gu,
                   beu, wu2, bu2, lng, lnb)
    t = _dot(hg, hw0[...]) + hb0[...]
    d, var = _bn0(t)
    t = jax.nn.relu(d / jnp.sqrt(var + 1e-5) * hg0[...] + hbe0[...])
    o_ref[...] = _dot(t, hw1[...]) + hb1[...]


@functools.partial(
    pl.kernel,
    out_type=[_sds((E, HID), _f32), _sds((NW, 8, LANES), _f32)],
    mesh=_sc_mesh,
    scratch_types=[
        pltpu.VMEM((CH,), jnp.int32),
        pltpu.VMEM((CH,), jnp.int32),
        pltpu.VMEM((CH, 2 * HID), _f32),
        pltpu.VMEM((CH, 2 * HID), _f32),
        pltpu.VMEM((CH, HID), _f32),
        pltpu.VMEM((CH, HID), _f32),
        pltpu.VMEM((8, LANES), _f32),
        pltpu.SemaphoreType.DMA,
        pltpu.SemaphoreType.DMA,
        pltpu.SemaphoreType.DMA,
    ],
)
def _sc_gather_stats(pq_hbm, src_hbm, dst_hbm, r_hbm, y_hbm, st_hbm,
                     sidx, didx, gp, gq, rb, yb, accv, s1, s2, s3):
    c = lax.axis_index("c")
    s = lax.axis_index("s")
    w = s * NC + c

    def chunk(j, accs):
        base = w * EPW + j * CH
        pltpu.sync_copy(src_hbm.at[pl.ds(base, CH)], sidx)
        pltpu.sync_copy(dst_hbm.at[pl.ds(base, CH)], didx)
        cp1 = pltpu.async_copy(pq_hbm.at[sidx], gp, s1)
        cp2 = pltpu.async_copy(pq_hbm.at[didx], gq, s2)
        cp3 = pltpu.async_copy(r_hbm.at[pl.ds(base, CH)], rb, s3)
        cp1.wait()
        cp2.wait()
        cp3.wait()

        def row(i, a):
            out = []
            for k in range(HID // LANES):
                sl = pl.ds(k * LANES, LANES)
                v = gp[i, pl.ds(k * LANES, LANES)] \
                    + gq[i, pl.ds(HID + k * LANES, LANES)] + rb[i, sl]
                yb[i, sl] = v
                out.append(a[2 * k] + v)
                out.append(a[2 * k + 1] + v * v)
            return tuple(out)

        accs = lax.fori_loop(0, CH, row, accs)
        pltpu.sync_copy(yb, y_hbm.at[pl.ds(base, CH)])
        return accs

    zero = jnp.zeros((LANES,), _f32)
    accs = lax.fori_loop(0, NCH, chunk, (zero,) * 8)
    for k in range(8):
        accv[k, :] = accs[k]
    pltpu.sync_copy(accv, st_hbm.at[w])


@functools.partial(
    pl.kernel,
    out_type=_sds((NC, NP, 2 * HID), _f32),
    mesh=_sc_mesh,
    scratch_types=[
        pltpu.VMEM((CH,), jnp.int32),
        pltpu.VMEM((CH, HID), _f32),
        pltpu.VMEM((CH, 2 * HID), _f32),
        pltpu.VMEM((8, LANES), _f32),
        pltpu.VMEM((RZ, 2 * HID), _f32),
        pltpu.VMEM_SHARED((NP, 2 * HID), _f32),
    ],
)
def _sc_norm_scatter(y_hbm, dst_hbm, ss_hbm, out_hbm, didx, yb, zb, ssv, zbuf,
                     acc_sh):
    c = lax.axis_index("c")
    s = lax.axis_index("s")
    w = s * NC + c
    pltpu.sync_copy(ss_hbm, ssv)
    ohv = jnp.where(lax.iota(jnp.int32, LANES) == 0, 1.0, 0.0).astype(_f32)

    def zrow(i, _):
        for k in range(2 * HID // LANES):
            zbuf[i, pl.ds(k * LANES, LANES)] = jnp.zeros((LANES,), _f32)
        return 0

    lax.fori_loop(0, RZ, zrow, 0)

    def zrow2(i, _):
        for k in range(2 * HID // LANES):
            zb[i, pl.ds(k * LANES, LANES)] = jnp.zeros((LANES,), _f32)
        zb[i, pl.ds(HID, LANES)] = ohv
        return 0

    lax.fori_loop(0, CH, zrow2, 0)

    def zcp(t, _):
        pltpu.sync_copy(zbuf, acc_sh.at[pl.ds(s * RPS + t * RZ, RZ)])
        return 0

    lax.fori_loop(0, NRC, zcp, 0)
    plsc.subcore_barrier()

    scs = [ssv[k, :] for k in range(4)]
    shs = [ssv[4 + k, :] for k in range(4)]

    def chunk(j, _):
        base = w * EPW + j * CH
        pltpu.sync_copy(y_hbm.at[pl.ds(base, CH)], yb)
        pltpu.sync_copy(dst_hbm.at[pl.ds(base, CH)], didx)

        def row(i, __):
            for k in range(HID // LANES):
                sl = pl.ds(k * LANES, LANES)
                z = jnp.maximum(yb[i, sl] * scs[k] + shs[k], 0.0)
                u = lax.bitcast_convert_type(z, jnp.uint32)
                u = (u + jnp.uint32(0x7FFF) + ((u >> 16) & jnp.uint32(1))) \
                    & jnp.uint32(0xFFFF0000)
                zb[i, sl] = lax.bitcast_convert_type(u, _f32)
            return 0

        lax.fori_loop(0, CH, row, 0)
        pltpu.sync_copy(zb, acc_sh.at[didx], add=True)
        return 0

    lax.fori_loop(0, NCH, chunk, 0)
    plsc.subcore_barrier()

    def rd(t, _):
        rb = s * RPS + t * RZ
        pltpu.sync_copy(acc_sh.at[pl.ds(rb, RZ)], zbuf)
        pltpu.sync_copy(zbuf, out_hbm.at[c, pl.ds(rb, RZ)])
        return 0

    lax.fori_loop(0, NRC, rd, 0)


def _full(shape):
    return pl.BlockSpec(shape, lambda i: (0,) * len(shape))


_EB = 3200
_NEB = E // _EB


def _call_node_enc(x, w0, b0, g0, be0, w1, b1, wab):
    return pl.pallas_call(
        _node_enc_body,
        out_shape=[_sds((N, HID), _f32), _sds((N, 2 * HID), _f32)],
    )(x, w0, b0, g0, be0, w1, b1, wab)


def _call_estats(a_pad, w0, b0):
    return pl.pallas_call(
        _estats_body,
        grid=(2, _NEB),
        in_specs=[
            pl.BlockSpec((_EB, 8), lambda ph, i: (i, 0)),
            pl.BlockSpec((8, EHID), lambda ph, i: (0, 0)),
            pl.BlockSpec((1, EHID), lambda ph, i: (0, 0)),
        ],
        out_specs=pl.BlockSpec((2, EHID), lambda ph, i: (0, 0)),
        out_shape=_sds((2, EHID), _f32),
        scratch_shapes=[pltpu.VMEM((2, EHID), _f32)],
    )(a_pad, w0, b0)


def _call_redge(a_pad, w0, b0, ss, ew1, eb1, w1cs, b1s):
    return pl.pallas_call(
        _redge_body,
        grid=(_NEB,),
        in_specs=[
            pl.BlockSpec((_EB, 8), lambda i: (i, 0)),
            _full((8, EHID)),
            _full((1, EHID)),
            _full((2, EHID)),
            _full((EHID, EHID)),
            _full((1, EHID)),
            _full((4 * EHID, HID)),
            _full((8, HID)),
        ],
        out_specs=[pl.BlockSpec((_EB, HID), lambda i: (i, 0))] * 4,
        out_shape=[_sds((E, HID), _f32)] * 4,
    )(a_pad, w0, b0, ss, ew1, eb1, w1cs, b1s)


def _call_upd(h, s0, s1, d0, d1, lw, wab):
    return pl.pallas_call(
        _upd_body,
        out_shape=[_sds((N, HID), _f32), _sds((N, 2 * HID), _f32)],
    )(h, s0, s1, d0, d1, *lw, wab)


def _call_upd_head(h, s0, s1, d0, d1, lw, hw):
    return pl.pallas_call(
        _upd_head_body,
        out_shape=_sds((N, 1), _f32),
    )(h, s0, s1, d0, d1, *lw, *hw)


def _row(v):
    return v.reshape(1, -1)


def kernel(x, edge_index, edge_attr, batch, params):
    src = edge_index[0]
    dst = edge_index[1]

    ne0, ne1 = params['node_enc']
    ee0, ee1 = params['edge_enc']
    layers = params['layers']
    he0, he1 = params['head']

    a_pad = jnp.pad(edge_attr, ((0, 0), (0, 4)))
    ew0p = jnp.pad(ee0['w'], ((0, 4), (0, 0)))

    w1s = [lp['msg'][0]['w'] for lp in layers]
    w1cs = jnp.concatenate([w1[128:160] for w1 in w1s], axis=0)
    b1s = jnp.stack([lp['msg'][0]['b'] for lp in layers], axis=0)
    b1s = jnp.pad(b1s, ((0, 4), (0, 0)))

    wabs = [jnp.concatenate([w1[:64], w1[64:128]], axis=1) for w1 in w1s]

    lws = []
    for lp in layers:
        u0, u1 = lp['upd']
        w2q = lp['msg'][1]['w'].astype(jnp.bfloat16).astype(_f32)
        lws.append((
            w2q, _row(lp['msg'][1]['b']),
            u0['w'][:64], u0['w'][64:], _row(u0['b']),
            _row(u0['g']), _row(u0['beta']),
            u1['w'], _row(u1['b']),
            _row(lp['ln_g']), _row(lp['ln_b']),
        ))
    hw = (he0['w'], _row(he0['b']), _row(he0['g']), _row(he0['beta']),
          he1['w'], _row(he1['b']))

    h, pq = _call_node_enc(
        x, ne0['w'], _row(ne0['b']), _row(ne0['g']), _row(ne0['beta']),
        ne1['w'], _row(ne1['b']), wabs[0])

    est = _call_estats(a_pad, ew0p, _row(ee0['b']))
    mu_e = est[0] / E
    var_e = est[1] / E
    sc_e = ee0['g'] / jnp.sqrt(var_e + 1e-5)
    ss_e = jnp.stack([sc_e, ee0['beta'] - mu_e * sc_e], axis=0)

    rs = _call_redge(a_pad, ew0p, _row(ee0['b']), ss_e, ee1['w'],
                     _row(ee1['b']), w1cs, b1s)

    out = None
    d0 = d1 = None
    for l in range(NLAYERS):
        y1, st = _sc_gather_stats(pq, src, dst, rs[l])
        sums = jnp.sum(st, axis=0)
        mu = sums[0::2].reshape(-1) / E
        ex2 = sums[1::2].reshape(-1) / E
        g1 = layers[l]['msg'][0]['g']
        be1 = layers[l]['msg'][0]['beta']
        scale = g1 / jnp.sqrt(ex2 - mu * mu + 1e-5)
        shift = be1 - mu * scale
        ss = jnp.concatenate([scale, shift]).reshape(8, LANES)
        sp = _sc_norm_scatter(y1, dst, ss)
        if l == 0:
            d0 = sp[0, :N, HID:HID + 1]
            d1 = sp[1, :N, HID:HID + 1]
        s0 = sp[0, :N, :HID]
        s1 = sp[1, :N, :HID]
        if l < NLAYERS - 1:
            h, pq = _call_upd(h, s0, s1, d0, d1, lws[l], wabs[l + 1])
        else:
            out = _call_upd_head(h, s0, s1, d0, d1, lws[l], hw)
    return out
